# Initial kernel scaffold; baseline (speedup 1.0000x reference)
import jax
import jax.numpy as jnp
from jax.experimental import pallas as pl


def kernel(enc_conv0_w, enc_bn0_g, enc_bn0_b, enc_b0_expand_w, enc_b0_conv1_w, enc_b0_bn1_g, enc_b0_bn1_b, enc_b0_conv2_w, enc_b0_bn2_g, enc_b0_bn2_b, enc_b1_expand_w, enc_b1_conv1_w, enc_b1_bn1_g, enc_b1_bn1_b, enc_b1_conv2_w, enc_b1_bn2_g, enc_b1_bn2_b, enc_b2_conv1_w, enc_b2_bn1_g, enc_b2_bn1_b, enc_b2_conv2_w, enc_b2_bn2_g, enc_b2_bn2_b, enc_fc_w, enc_fc_b, dec_fc_w, dec_fc_b, dec_b0_conv1_w, dec_b0_bn1_g, dec_b0_bn1_b, dec_b0_conv2_w, dec_b0_bn2_g, dec_b0_bn2_b, dec_b1_expand_w, dec_b1_conv1_w, dec_b1_bn1_g, dec_b1_bn1_b, dec_b1_conv2_w, dec_b1_bn2_g, dec_b1_bn2_b, dec_b2_expand_w, dec_b2_conv1_w, dec_b2_bn1_g, dec_b2_bn1_b, dec_b2_conv2_w, dec_b2_bn2_g, dec_b2_bn2_b, dec_b3_conv1_w, dec_b3_bn1_g, dec_b3_bn1_b, dec_b3_conv2_w, dec_b3_bn2_g, dec_b3_bn2_b, dec_pred_w, dec_pred_b, x, noise_key):
    raise NotImplementedError("write your pallas kernel here")


# speedup vs baseline: 1.4232x; 1.4232x over previous
import functools

import jax
import jax.numpy as jnp
from jax.experimental import pallas as pl
from jax.experimental.pallas import tpu as pltpu

_VMEM = 64 * 1024 * 1024
_RAW = jnp.float32
_CORES = 1


def _cp(sem):
    return pltpu.CompilerParams(dimension_semantics=sem,
                                vmem_limit_bytes=_VMEM)


def _full(shape):
    return pl.BlockSpec(shape, lambda *_: (0,) * len(shape))


def _stage_body(*refs, N, Hi, Wi, Wpi, Cin, bn, has_res, mode, conv, kh, kw,
                Cout, has_expand, has_clean, has_bias, sig, stats,
                preflat=False, flat_for=None, slope=0.2, eps=1e-5):
    it = iter(refs)
    in_ref = next(it)
    if bn:
        st_ref = next(it)
        gb_ref = next(it)
    if has_res:
        res_ref = next(it)
    if conv:
        w_ref = next(it)
    if has_expand:
        ew_ref = next(it)
    if has_bias:
        b_ref = next(it)
    if flat_for is not None:
        f_ref = next(it)
    if conv:
        y_ref = next(it)
    if stats:
        so_ref = next(it)
    if has_expand:
        e_ref = next(it)
    if has_clean:
        c_ref = next(it)

    if bn:
        y = in_ref[0].reshape(Hi, Wpi, Cin)[:, 0:Wi, :].astype(jnp.float32)
        st = st_ref[...]
        s = jnp.sum(st[:, 0, :], axis=0, keepdims=True)
        ss = jnp.sum(st[:, 1, :], axis=0, keepdims=True)
        inv = 1.0 / float(N * Hi * Wi)
        mean = s * inv
        var = ss * inv - mean * mean
        scale = gb_ref[0:1, :] * jax.lax.rsqrt(var + eps)
        shift = gb_ref[1:2, :] - mean * scale
        a = y * scale.reshape(1, 1, Cin) + shift.reshape(1, 1, Cin)
        if has_res:
            a = a + res_ref[0].reshape(Hi, Wi, Cin).astype(jnp.float32)
        a = jnp.where(a >= 0.0, a, slope * a)
    elif preflat:
        a = None
    else:
        a = in_ref[0].astype(jnp.float32)

    if mode == "pool":
        Ho, Wo = Hi // 2, Wi // 2
        ar = a.reshape(Ho, 2, Wi, Cin)
        rowsum = ar[:, 0] + ar[:, 1]
        pm = (jax.lax.broadcasted_iota(jnp.int32, (Wo, Wi), 1) // 2
              == jax.lax.broadcasted_iota(jnp.int32, (Wo, Wi), 0)
              ).astype(jnp.float32)
        a = jnp.stack([0.25 * jnp.dot(pm, rowsum[h],
                                      preferred_element_type=jnp.float32)
                       for h in range(Ho)], axis=0)
    elif mode == "up":
        Ho, Wo = 2 * Hi, 2 * Wi
        um = (jax.lax.broadcasted_iota(jnp.int32, (Wo, Wi), 0) // 2
              == jax.lax.broadcasted_iota(jnp.int32, (Wo, Wi), 1)
              ).astype(jnp.float32)
        wide = jnp.stack([jnp.dot(um, a[h],
                                  preferred_element_type=jnp.float32)
                          for h in range(Hi)], axis=0)
        a = jnp.broadcast_to(wide.reshape(Hi, 1, Wo, Cin),
                             (Hi, 2, Wo, Cin)).reshape(Ho, Wo, Cin)
    else:
        Ho, Wo = Hi, Wi

    if has_clean:
        c_ref[0] = a.astype(c_ref.dtype)

    ab = a.astype(jnp.bfloat16) if a is not None else None

    if has_expand:
        e_ref[0] = jnp.dot(ab.reshape(Ho * Wo, Cin), ew_ref[...],
                           preferred_element_type=jnp.float32
                           ).astype(e_ref.dtype)

    if flat_for is not None:
        fh, fw = flat_for
        ph, pw = (fh - 1) // 2, (fw - 1) // 2
        ap = jnp.pad(ab, ((ph, ph), (pw, pw), (0, 0)))
        fl = ap.reshape((Ho + 2 * ph) * (Wo + 2 * pw), Cin)
        if fw > 1:
            fl = jnp.pad(fl, ((0, fw - 1), (0, 0)))
        f_ref[0] = fl
        return

    if not conv:
        return

    ph, pw = (kh - 1) // 2, (kw - 1) // 2
    Wp = Wo + 2 * pw
    flat = in_ref[0]
    L = Ho * Wp
    acc = None
    for dy in range(kh):
        for dx in range(kw):
            off = dy * Wp + dx
            xs = flat[off:off + L, :]
            wk = w_ref[dy * kw + dx]
            t = jnp.dot(xs, wk, preferred_element_type=jnp.float32)
            acc = t if acc is None else acc + t
    if has_bias:
        acc = acc + b_ref[...]
    if sig:
        acc = 1.0 / (1.0 + jnp.exp(-acc))
    y_ref[0] = acc.astype(y_ref.dtype)

    if stats:
        ridx = jax.lax.broadcasted_iota(jnp.int32, (L, 1), 0)
        valid = (ridx % Wp) < Wo
        av = jnp.where(valid, acc, 0.0)
        so_ref[0, 0:1, :] = jnp.sum(av, axis=0, keepdims=True)
        so_ref[0, 1:2, :] = jnp.sum(av * av, axis=0, keepdims=True)


def _stage(x_in, *, Hi, Wi, Wpi=None, st=None, gamma=None, beta=None,
           res=None, mode="none", w=None, expand_w=None, bias=None,
           sig=False, clean_dt=None, raw_dt=None, y_dt=None, preflat=False,
           flat_for=None):
    N = x_in.shape[0]
    bn = st is not None
    conv = w is not None
    Cin = x_in.shape[-1] if not bn else x_in.shape[-1]
    raw_dt = raw_dt if raw_dt is not None else _RAW

    N2 = N // _CORES

    def _img(ndim):
        return lambda c, i: (c * N2 + i,) + (0,) * (ndim - 1)

    args = [x_in]
    in_specs = [pl.BlockSpec((1,) + x_in.shape[1:], _img(x_in.ndim))]
    if bn:
        gb = jnp.stack([gamma, beta], axis=0).astype(jnp.float32)
        args += [st, gb]
        in_specs += [_full(st.shape), _full(gb.shape)]
    if res is not None:
        args.append(res)
        in_specs.append(pl.BlockSpec((1,) + res.shape[1:], _img(res.ndim)))
    kh = kw = 0
    Cout = 0
    if conv:
        Cout, Cin2, kh, kw = w.shape
        wk = jnp.transpose(w, (2, 3, 1, 0)).reshape(kh * kw, Cin2, Cout)
        args.append(wk.astype(jnp.bfloat16))
        in_specs.append(_full((kh * kw, Cin2, Cout)))
    if expand_w is not None:
        Cexp = expand_w.shape[0]
        ew = expand_w.reshape(Cexp, expand_w.shape[1]).T
        args.append(ew.astype(jnp.bfloat16))
        in_specs.append(_full(ew.shape))
    if bias is not None:
        args.append(bias.reshape(1, -1).astype(jnp.float32))
        in_specs.append(_full((1, bias.size)))

    if mode == "pool":
        Ho, Wo = Hi // 2, Wi // 2
    elif mode == "up":
        Ho, Wo = 2 * Hi, 2 * Wi
    else:
        Ho, Wo = Hi, Wi

    stats = conv and not sig
    out_shapes, out_specs = [], []
    if flat_for is not None:
        fh, fw = flat_for
        fph, fpw = (fh - 1) // 2, (fw - 1) // 2
        Lpad = (Ho + 2 * fph) * (Wo + 2 * fpw) + (fw - 1 if fw > 1 else 0)
        out_shapes.append(jax.ShapeDtypeStruct((N, Lpad, Cin), jnp.bfloat16))
        out_specs.append(pl.BlockSpec((1, Lpad, Cin), _img(3)))
    if conv:
        ph, pw = (kh - 1) // 2, (kw - 1) // 2
        Wp = Wo + 2 * pw
        L = Ho * Wp
        out_shapes.append(jax.ShapeDtypeStruct((N, L, Cout),
                                               y_dt if y_dt else raw_dt))
        out_specs.append(pl.BlockSpec((1, L, Cout), _img(3)))
        if stats:
            out_shapes.append(jax.ShapeDtypeStruct((N, 2, Cout), jnp.float32))
            out_specs.append(pl.BlockSpec((1, 2, Cout), _img(3)))
    if expand_w is not None:
        out_shapes.append(jax.ShapeDtypeStruct((N, Ho * Wo, Cexp), raw_dt))
        out_specs.append(pl.BlockSpec((1, Ho * Wo, Cexp), _img(3)))
    if clean_dt is not None:
        out_shapes.append(jax.ShapeDtypeStruct((N, Ho, Wo, Cin), clean_dt))
        out_specs.append(pl.BlockSpec((1, Ho, Wo, Cin), _img(4)))

    body = functools.partial(
        _stage_body, N=N, Hi=Hi, Wi=Wi, Wpi=Wpi, Cin=Cin, bn=bn,
        has_res=res is not None, mode=mode, conv=conv, kh=kh, kw=kw,
        Cout=Cout, has_expand=expand_w is not None,
        has_clean=clean_dt is not None, has_bias=bias is not None,
        sig=sig, stats=stats, preflat=preflat, flat_for=flat_for)

    outs = pl.pallas_call(
        body, grid=(_CORES, N2), in_specs=in_specs,
        out_specs=tuple(out_specs) if len(out_specs) > 1 else out_specs[0],
        out_shape=tuple(out_shapes) if len(out_shapes) > 1 else out_shapes[0],
        compiler_params=_cp(("core_parallel", "arbitrary")),
    )(*args)
    if len(out_shapes) == 1:
        outs = (outs,)
    outs = list(outs)

    raw = st_out = exp_out = clean = None
    i = 0
    if flat_for is not None:
        raw = outs[i]; i += 1
    if conv:
        raw = outs[i]; i += 1
        if stats:
            st_out = outs[i]; i += 1
    if expand_w is not None:
        exp_out = outs[i]; i += 1
    if clean_dt is not None:
        clean = outs[i]; i += 1
    return raw, st_out, exp_out, clean


def _pad_flat(x_nhwc, kh, kw):
    N, H, W, C = x_nhwc.shape
    ph, pw = (kh - 1) // 2, (kw - 1) // 2
    xp = jnp.pad(x_nhwc.astype(jnp.bfloat16),
                 ((0, 0), (ph, ph), (pw, pw), (0, 0)))
    flat = xp.reshape(N, (H + 2 * ph) * (W + 2 * pw), C)
    if kw > 1:
        flat = jnp.pad(flat, ((0, 0), (0, kw - 1), (0, 0)))
    return flat


def _fc_body(yf_ref, we_ref, be_ref, eps_ref, wd_ref, bd_ref,
             fc_ref, z_ref, h_ref, *, zdim):
    fc = jnp.dot(yf_ref[...], we_ref[...],
                 preferred_element_type=jnp.float32) + be_ref[...]
    fc_ref[...] = fc
    mu = fc[:, 0:zdim]
    lv = fc[:, zdim:2 * zdim]
    z = mu + eps_ref[...] * jnp.exp(0.5 * lv)
    z_ref[...] = z
    h = jnp.dot(z.astype(jnp.bfloat16), wd_ref[...],
                preferred_element_type=jnp.float32) + bd_ref[...]
    h_ref[...] = jnp.maximum(h, 0.0)


def _fc_block(yf, enc_fc_w, enc_fc_b, eps, dec_fc_w, dec_fc_b, zdim):
    M, K = yf.shape
    Nout = enc_fc_w.shape[0]
    Kd = dec_fc_w.shape[0]
    we = jnp.transpose(enc_fc_w).astype(jnp.bfloat16)
    wd = jnp.transpose(dec_fc_w).astype(jnp.bfloat16)
    fc, z, h = pl.pallas_call(
        functools.partial(_fc_body, zdim=zdim),
        grid=(1,),
        in_specs=[_full((M, K)), _full((K, Nout)), _full((1, Nout)),
                  _full((M, zdim)), _full((zdim, Kd)), _full((1, Kd))],
        out_specs=(_full((M, Nout)), _full((M, zdim)), _full((M, Kd))),
        out_shape=(jax.ShapeDtypeStruct((M, Nout), jnp.float32),
                   jax.ShapeDtypeStruct((M, zdim), jnp.float32),
                   jax.ShapeDtypeStruct((M, Kd), jnp.float32)),
        compiler_params=_cp(("arbitrary",)),
    )(yf.astype(jnp.bfloat16), we,
      enc_fc_b.reshape(1, Nout).astype(jnp.float32), eps.astype(jnp.float32),
      wd, dec_fc_b.reshape(1, Kd).astype(jnp.float32))
    return fc, z, h


def kernel(enc_conv0_w, enc_bn0_g, enc_bn0_b, enc_b0_expand_w, enc_b0_conv1_w,
           enc_b0_bn1_g, enc_b0_bn1_b, enc_b0_conv2_w, enc_b0_bn2_g,
           enc_b0_bn2_b, enc_b1_expand_w, enc_b1_conv1_w, enc_b1_bn1_g,
           enc_b1_bn1_b, enc_b1_conv2_w, enc_b1_bn2_g, enc_b1_bn2_b,
           enc_b2_conv1_w, enc_b2_bn1_g, enc_b2_bn1_b, enc_b2_conv2_w,
           enc_b2_bn2_g, enc_b2_bn2_b, enc_fc_w, enc_fc_b, dec_fc_w, dec_fc_b,
           dec_b0_conv1_w, dec_b0_bn1_g, dec_b0_bn1_b, dec_b0_conv2_w,
           dec_b0_bn2_g, dec_b0_bn2_b, dec_b1_expand_w, dec_b1_conv1_w,
           dec_b1_bn1_g, dec_b1_bn1_b, dec_b1_conv2_w, dec_b1_bn2_g,
           dec_b1_bn2_b, dec_b2_expand_w, dec_b2_conv1_w, dec_b2_bn1_g,
           dec_b2_bn1_b, dec_b2_conv2_w, dec_b2_bn2_g, dec_b2_bn2_b,
           dec_b3_conv1_w, dec_b3_bn1_g, dec_b3_bn1_b, dec_b3_conv2_w,
           dec_b3_bn2_g, dec_b3_bn2_b, dec_pred_w, dec_pred_b, x, noise_key):
    N, cdim, H, W = x.shape
    zdim = dec_fc_w.shape[1]
    x_nhwc = jnp.transpose(x.astype(jnp.float32), (0, 2, 3, 1))

    r0, s0, _, _ = _stage(_pad_flat(x_nhwc, 5, 5), Hi=H, Wi=W,
                          w=enc_conv0_w, preflat=True)
    c0 = enc_conv0_w.shape[0]

    f1, _, id0, e0 = _stage(
        r0, Hi=H, Wi=W, Wpi=W + 4, st=s0, gamma=enc_bn0_g, beta=enc_bn0_b,
        mode="pool", flat_for=(3, 3), expand_w=enc_b0_expand_w,
        clean_dt=jnp.float32)
    H1, W1 = H // 2, W // 2
    c1 = enc_b0_conv1_w.shape[0]
    r1, s1, _, _ = _stage(f1, Hi=H1, Wi=W1, w=enc_b0_conv1_w, preflat=True)

    f2, _, _, _ = _stage(r1, Hi=H1, Wi=W1, Wpi=W1 + 2, st=s1,
                         gamma=enc_b0_bn1_g, beta=enc_b0_bn1_b,
                         flat_for=(3, 3))
    r2, s2, _, _ = _stage(f2, Hi=H1, Wi=W1, w=enc_b0_conv2_w, preflat=True)

    f3, _, id1, e1 = _stage(
        r2, Hi=H1, Wi=W1, Wpi=W1 + 2, st=s2, gamma=enc_b0_bn2_g,
        beta=enc_b0_bn2_b, res=id0, mode="pool", flat_for=(3, 3),
        expand_w=enc_b1_expand_w, clean_dt=jnp.float32)
    H2, W2 = H1 // 2, W1 // 2
    c2 = enc_b1_conv1_w.shape[0]
    r3, s3, _, _ = _stage(f3, Hi=H2, Wi=W2, w=enc_b1_conv1_w, preflat=True)

    f4, _, _, _ = _stage(r3, Hi=H2, Wi=W2, Wpi=W2 + 2, st=s3,
                         gamma=enc_b1_bn1_g, beta=enc_b1_bn1_b,
                         flat_for=(3, 3))
    r4, s4, _, _ = _stage(f4, Hi=H2, Wi=W2, w=enc_b1_conv2_w, preflat=True)

    f5, _, _, e2 = _stage(
        r4, Hi=H2, Wi=W2, Wpi=W2 + 2, st=s4, gamma=enc_b1_bn2_g,
        beta=enc_b1_bn2_b, res=id1, mode="pool", flat_for=(3, 3),
        clean_dt=jnp.float32)
    H3, W3 = H2 // 2, W2 // 2
    r5, s5, _, _ = _stage(f5, Hi=H3, Wi=W3, w=enc_b2_conv1_w, preflat=True)

    f6, _, _, _ = _stage(r5, Hi=H3, Wi=W3, Wpi=W3 + 2, st=s5,
                         gamma=enc_b2_bn1_g, beta=enc_b2_bn1_b,
                         flat_for=(3, 3))
    r6, s6, _, _ = _stage(f6, Hi=H3, Wi=W3, w=enc_b2_conv2_w, preflat=True)

    e2_res = e2.reshape(N, H3 * W3, c2)
    _, _, _, yf_img = _stage(r6, Hi=H3, Wi=W3, Wpi=W3 + 2, st=s6,
                             gamma=enc_b2_bn2_g, beta=enc_b2_bn2_b,
                             res=e2_res, clean_dt=jnp.float32)

    yf = jnp.transpose(yf_img, (0, 3, 1, 2)).reshape(N, c2 * H3 * W3)

    eps = jax.random.normal(noise_key, (N, zdim), jnp.float32)
    fc, z, hdec = _fc_block(yf, enc_fc_w, enc_fc_b, eps, dec_fc_w, dec_fc_b,
                            zdim)
    mu = fc[:, :zdim]
    logvar = fc[:, zdim:]
    h0 = jnp.transpose(hdec.reshape(N, c2, H3, W3), (0, 2, 3, 1))

    d0, t0, _, _ = _stage(_pad_flat(h0, 3, 3), Hi=H3, Wi=W3,
                          w=dec_b0_conv1_w, preflat=True)
    g1, _, _, _ = _stage(d0, Hi=H3, Wi=W3, Wpi=W3 + 2, st=t0,
                         gamma=dec_b0_bn1_g, beta=dec_b0_bn1_b,
                         flat_for=(3, 3))
    d1, t1, _, _ = _stage(g1, Hi=H3, Wi=W3, w=dec_b0_conv2_w, preflat=True)

    h0_res = h0.reshape(N, H3 * W3, c2)
    g2, _, jd1, _ = _stage(
        d1, Hi=H3, Wi=W3, Wpi=W3 + 2, st=t1, gamma=dec_b0_bn2_g,
        beta=dec_b0_bn2_b, res=h0_res, mode="up", flat_for=(3, 3),
        expand_w=dec_b1_expand_w)
    d2, t2, _, _ = _stage(g2, Hi=H2, Wi=W2, w=dec_b1_conv1_w, preflat=True)

    g3, _, _, _ = _stage(d2, Hi=H2, Wi=W2, Wpi=W2 + 2, st=t2,
                         gamma=dec_b1_bn1_g, beta=dec_b1_bn1_b,
                         flat_for=(3, 3))
    d3, t3, _, _ = _stage(g3, Hi=H2, Wi=W2, w=dec_b1_conv2_w, preflat=True)

    g4, _, jd2, _ = _stage(
        d3, Hi=H2, Wi=W2, Wpi=W2 + 2, st=t3, gamma=dec_b1_bn2_g,
        beta=dec_b1_bn2_b, res=jd1, mode="up", flat_for=(3, 3),
        expand_w=dec_b2_expand_w)
    d4, t4, _, _ = _stage(g4, Hi=H1, Wi=W1, w=dec_b2_conv1_w, preflat=True)

    g5, _, _, _ = _stage(d4, Hi=H1, Wi=W1, Wpi=W1 + 2, st=t4,
                         gamma=dec_b2_bn1_g, beta=dec_b2_bn1_b,
                         flat_for=(3, 3))
    d5, t5, _, _ = _stage(g5, Hi=H1, Wi=W1, w=dec_b2_conv2_w, preflat=True)

    g6, _, _, a3 = _stage(
        d5, Hi=H1, Wi=W1, Wpi=W1 + 2, st=t5, gamma=dec_b2_bn2_g,
        beta=dec_b2_bn2_b, res=jd2, mode="up", flat_for=(3, 3),
        clean_dt=_RAW)
    d6, t6, _, _ = _stage(g6, Hi=H, Wi=W, w=dec_b3_conv1_w, preflat=True)

    g7, _, _, _ = _stage(d6, Hi=H, Wi=W, Wpi=W + 2, st=t6,
                         gamma=dec_b3_bn1_g, beta=dec_b3_bn1_b,
                         flat_for=(3, 3))
    d7, t7, _, _ = _stage(g7, Hi=H, Wi=W, w=dec_b3_conv2_w, preflat=True)

    c3 = dec_b3_conv1_w.shape[0]
    a3_res = a3.reshape(N, H * W, c3)
    g8, _, _, _ = _stage(
        d7, Hi=H, Wi=W, Wpi=W + 2, st=t7, gamma=dec_b3_bn2_g,
        beta=dec_b3_bn2_b, res=a3_res, flat_for=(5, 5))
    pred, _, _, _ = _stage(g8, Hi=H, Wi=W, w=dec_pred_w, bias=dec_pred_b,
                           sig=True, y_dt=jnp.float32, preflat=True)

    xr = pred.reshape(N, H, W + 4, cdim)[:, :, :W, :]
    y = jnp.transpose(xr, (0, 3, 1, 2))

    emb = [jnp.transpose(e0, (0, 3, 1, 2)),
           jnp.transpose(e1, (0, 3, 1, 2)),
           jnp.transpose(e2, (0, 3, 1, 2))]
    return y, {"x_rec": y, "z_mu": mu, "z_logvar": logvar, "z": z,
               "embeddings": emb}
